# baseline (speedup 1.0000x reference)
import jax
import jax.numpy as jnp
from jax.experimental import pallas as pl
from jax.experimental.pallas import tpu as pltpu

_NCHUNKS = 8


def _dma_body(t_ref, o_ref, sems):
    rows = o_ref.shape[0]
    chunk = rows // _NCHUNKS
    copies = [
        pltpu.make_async_copy(
            t_ref.at[pl.ds(i * chunk, chunk), :],
            o_ref.at[pl.ds(i * chunk, chunk), :],
            sems.at[i],
        )
        for i in range(_NCHUNKS)
    ]
    for c in copies:
        c.start()
    for c in copies:
        c.wait()


def kernel(x, table):
    seq_len = x.shape[1]
    dim = table.shape[1]
    return pl.pallas_call(
        _dma_body,
        in_specs=[pl.BlockSpec(memory_space=pl.ANY)],
        out_specs=pl.BlockSpec(memory_space=pl.ANY),
        out_shape=jax.ShapeDtypeStruct((seq_len, dim), table.dtype),
        scratch_shapes=[pltpu.SemaphoreType.DMA((_NCHUNKS,))],
    )(table)


# speedup vs baseline: 29.1434x; 29.1434x over previous
import jax
import jax.numpy as jnp
from jax.experimental import pallas as pl

_BLOCK_ROWS = 256


def _copy_body(t_ref, o_ref):
    o_ref[...] = t_ref[...]


def kernel(x, table):
    seq_len = x.shape[1]
    dim = table.shape[1]
    return pl.pallas_call(
        _copy_body,
        grid=(seq_len // _BLOCK_ROWS,),
        in_specs=[pl.BlockSpec((_BLOCK_ROWS, dim), lambda i: (i, 0))],
        out_specs=pl.BlockSpec((_BLOCK_ROWS, dim), lambda i: (i, 0)),
        out_shape=jax.ShapeDtypeStruct((seq_len, dim), table.dtype),
    )(table)


# speedup vs baseline: 41.9930x; 1.4409x over previous
import jax
import jax.numpy as jnp
from jax.experimental import pallas as pl

_BLOCK_ROWS = 1024


def _copy_body(t_ref, o_ref):
    o_ref[...] = t_ref[...]


def kernel(x, table):
    seq_len = x.shape[1]
    dim = table.shape[1]
    return pl.pallas_call(
        _copy_body,
        grid=(seq_len // _BLOCK_ROWS,),
        in_specs=[pl.BlockSpec((_BLOCK_ROWS, dim), lambda i: (i, 0))],
        out_specs=pl.BlockSpec((_BLOCK_ROWS, dim), lambda i: (i, 0)),
        out_shape=jax.ShapeDtypeStruct((seq_len, dim), table.dtype),
    )(table)


# speedup vs baseline: 47.1558x; 1.1229x over previous
import jax
import jax.numpy as jnp
from jax.experimental import pallas as pl

_BLOCK_ROWS = 2048


def _copy_body(t_ref, o_ref):
    o_ref[...] = t_ref[...]


def kernel(x, table):
    seq_len = x.shape[1]
    dim = table.shape[1]
    return pl.pallas_call(
        _copy_body,
        grid=(seq_len // _BLOCK_ROWS,),
        in_specs=[pl.BlockSpec((_BLOCK_ROWS, dim), lambda i: (i, 0))],
        out_specs=pl.BlockSpec((_BLOCK_ROWS, dim), lambda i: (i, 0)),
        out_shape=jax.ShapeDtypeStruct((seq_len, dim), table.dtype),
    )(table)
